# Initial kernel scaffold; baseline (speedup 1.0000x reference)
import jax
import jax.numpy as jnp
from jax.experimental import pallas as pl


def kernel(x_nchw, w3, b3, gamma, beta, w1, b1):
    raise NotImplementedError("write your pallas kernel here")


# speedup vs baseline: 1.9357x; 1.9357x over previous
import math
from functools import partial

import jax
import jax.numpy as jnp
from jax import lax
from jax.experimental import pallas as pl
from jax.experimental.pallas import tpu as pltpu

EPS = 1e-5
WT = 4
GIMG = 8


def _band3_tile(w3, wt):
    cin, cout = w3.shape[2], w3.shape[3]
    pi = jnp.arange(wt + 2)[:, None]
    po = jnp.arange(wt)[None, :]
    kx = pi - po
    valid = ((kx >= 0) & (kx < 3)).astype(w3.dtype)
    g = w3[:, jnp.clip(kx, 0, 2)]
    g = g * valid[None, :, :, None, None]
    g = jnp.transpose(g, (0, 1, 3, 2, 4))
    return g.reshape(3, (wt + 2) * cin, wt * cout)


def _band1_tile(w1, wt):
    cin, cout = w1.shape
    pi = jnp.arange(wt + 2)[:, None]
    po = jnp.arange(wt)[None, :]
    sel = (pi == po + 1).astype(w1.dtype)
    g = sel[:, None, :, None] * w1[None, :, None, :]
    return g.reshape((wt + 2) * cin, wt * cout)


def _tile_lanes(v, w):
    return jnp.tile(v.reshape(1, -1), (1, w))


def _p1_kernel(x_ref, w_ref, b3_ref, y1_ref, st_ref, *, H, W, Cin, Cout):
    x = x_ref[0]
    gh = x.shape[0]
    zpad = jnp.zeros((gh, Cin), jnp.float32)
    xp = jnp.concatenate([zpad, x, zpad], axis=1)
    zrow = jnp.zeros((1, xp.shape[1]), jnp.float32)
    row = lax.broadcasted_iota(jnp.int32, (gh, 1), 0) % H
    xup = jnp.where(row == 0, 0.0, jnp.concatenate([zrow, xp[:gh - 1]], 0))
    xdn = jnp.where(row == H - 1, 0.0, jnp.concatenate([xp[1:], zrow], 0))
    kt = (WT + 2) * Cin
    ot = WT * Cout
    sums, sqs = [], []
    for t in range(W // WT):
        si = t * WT * Cin
        so = t * WT * Cout
        a = (jnp.dot(xup[:, si:si + kt], w_ref[0],
                     preferred_element_type=jnp.float32)
             + jnp.dot(xp[:, si:si + kt], w_ref[1],
                       preferred_element_type=jnp.float32)
             + jnp.dot(xdn[:, si:si + kt], w_ref[2],
                       preferred_element_type=jnp.float32))
        y = a + b3_ref[:, so:so + ot]
        y1_ref[0, :, so:so + ot] = y
        sums.append(jnp.sum(y, axis=0, keepdims=True))
        sqs.append(jnp.sum(y * y, axis=0, keepdims=True))
    st_ref[0] = jnp.concatenate(
        [jnp.concatenate(sums, axis=1), jnp.concatenate(sqs, axis=1)], axis=0)


def _p2_kernel(x_ref, y1_ref, w1_ref, b1_ref, sc_ref, sh_ref, o_ref,
               *, W, Cin, Cout):
    x = x_ref[0]
    gh = x.shape[0]
    zpad = jnp.zeros((gh, Cin), jnp.float32)
    xp = jnp.concatenate([zpad, x, zpad], axis=1)
    kt = (WT + 2) * Cin
    ot = WT * Cout
    for t in range(W // WT):
        si = t * WT * Cin
        so = t * WT * Cout
        y2 = jnp.dot(xp[:, si:si + kt], w1_ref[...],
                     preferred_element_type=jnp.float32)
        y1n = jnp.maximum(
            y1_ref[0, :, so:so + ot] * sc_ref[:, so:so + ot]
            + sh_ref[:, so:so + ot], 0.0)
        o_ref[0, :, so:so + ot] = jnp.maximum(
            y1n + y2 + b1_ref[:, so:so + ot], 0.0)


@jax.jit
def _forward(x_nchw, w3, b3, gamma, beta, w1, b1):
    N, Cin, H, W = x_nchw.shape
    Cout = w3.shape[-1]
    WCin, WCout = W * Cin, W * Cout
    P = N * H * W
    g = math.gcd(GIMG, N)
    ng = N // g
    gh = g * H

    x = jnp.transpose(x_nchw, (0, 2, 3, 1)).reshape(ng, gh, WCin)
    x = x.astype(jnp.float32)
    w3b = _band3_tile(w3.astype(jnp.float32), WT)
    w1b = _band1_tile(w1.astype(jnp.float32), WT)
    b3t = _tile_lanes(b3, W).astype(jnp.float32)
    b1t = _tile_lanes(b1, W).astype(jnp.float32)

    cparams = pltpu.CompilerParams(
        dimension_semantics=("parallel",),
        vmem_limit_bytes=64 * 1024 * 1024,
    )

    kt = (WT + 2) * Cin
    ot = WT * Cout
    nt = W // WT

    flops1 = int(ng * nt * 3 * gh * kt * ot * 2 + N * 6 * H * WCout)
    bytes1 = int(4 * (N * H * WCin + 3 * kt * ot + WCout
                      + N * H * WCout + ng * 2 * WCout))
    y1, stats = pl.pallas_call(
        partial(_p1_kernel, H=H, W=W, Cin=Cin, Cout=Cout),
        grid=(ng,),
        in_specs=[
            pl.BlockSpec((1, gh, WCin), lambda n: (n, 0, 0)),
            pl.BlockSpec((3, kt, ot), lambda n: (0, 0, 0)),
            pl.BlockSpec((1, WCout), lambda n: (0, 0)),
        ],
        out_specs=(
            pl.BlockSpec((1, gh, WCout), lambda n: (n, 0, 0)),
            pl.BlockSpec((1, 2, WCout), lambda n: (n, 0, 0)),
        ),
        out_shape=(
            jax.ShapeDtypeStruct((ng, gh, WCout), jnp.float32),
            jax.ShapeDtypeStruct((ng, 2, WCout), jnp.float32),
        ),
        compiler_params=cparams,
        cost_estimate=pl.CostEstimate(flops=flops1, transcendentals=0,
                                      bytes_accessed=bytes1),
    )(x, w3b, b3t)

    s = stats.sum(axis=0).reshape(2, W, Cout).sum(axis=1)
    mean = s[0] / P
    var = s[1] / P - mean * mean
    scale = gamma.reshape(Cout) * lax.rsqrt(var + EPS)
    shift = beta.reshape(Cout) - mean * scale
    sc = _tile_lanes(scale, W).astype(jnp.float32)
    sh = _tile_lanes(shift, W).astype(jnp.float32)

    flops2 = int(ng * nt * gh * kt * ot * 2 + N * 6 * H * WCout)
    bytes2 = int(4 * (N * H * WCin + N * H * WCout + kt * ot + 3 * WCout
                      + N * H * WCout))
    out = pl.pallas_call(
        partial(_p2_kernel, W=W, Cin=Cin, Cout=Cout),
        grid=(ng,),
        in_specs=[
            pl.BlockSpec((1, gh, WCin), lambda n: (n, 0, 0)),
            pl.BlockSpec((1, gh, WCout), lambda n: (n, 0, 0)),
            pl.BlockSpec((kt, ot), lambda n: (0, 0)),
            pl.BlockSpec((1, WCout), lambda n: (0, 0)),
            pl.BlockSpec((1, WCout), lambda n: (0, 0)),
            pl.BlockSpec((1, WCout), lambda n: (0, 0)),
        ],
        out_specs=pl.BlockSpec((1, gh, WCout), lambda n: (n, 0, 0)),
        out_shape=jax.ShapeDtypeStruct((ng, gh, WCout), jnp.float32),
        compiler_params=cparams,
        cost_estimate=pl.CostEstimate(flops=flops2, transcendentals=0,
                                      bytes_accessed=bytes2),
    )(x, y1, w1b, b1t, sc, sh)

    out = out.reshape(N, H, W, Cout)
    return jnp.transpose(out, (0, 3, 1, 2))


def kernel(x_nchw, w3, b3, gamma, beta, w1, b1):
    return _forward(x_nchw, w3, b3, gamma, beta, w1, b1)


# speedup vs baseline: 2.0591x; 1.0637x over previous
import math
from functools import partial

import jax
import jax.numpy as jnp
from jax import lax
from jax.experimental import pallas as pl
from jax.experimental.pallas import tpu as pltpu

EPS = 1e-5
WT = 4
NPI = 8
GIMG = 8


def _band3_tile(w3, off):
    cin, cout = w3.shape[2], w3.shape[3]
    pi = jnp.arange(NPI)[:, None]
    po = jnp.arange(WT)[None, :]
    kx = pi - po + 1 - off
    valid = ((kx >= 0) & (kx < 3)).astype(w3.dtype)
    g = w3[:, jnp.clip(kx, 0, 2)]
    g = g * valid[None, :, :, None, None]
    g = jnp.transpose(g, (0, 1, 3, 2, 4))
    return g.reshape(3, NPI * cin, WT * cout)


def _band1_tile(w1, off):
    cin, cout = w1.shape
    pi = jnp.arange(NPI)[:, None]
    po = jnp.arange(WT)[None, :]
    sel = (pi == po + off).astype(w1.dtype)
    g = sel[:, None, :, None] * w1[None, :, None, :]
    return g.reshape(NPI * cin, WT * cout)


def _tile_lanes(v, w):
    return jnp.tile(v.reshape(1, -1), (1, w))


def _p1_kernel(x_ref, w_ref, b3_ref, y1_ref, st_ref, *, H, W, Cin, Cout):
    x = x_ref[0]
    gh = x.shape[0]
    zrow = jnp.zeros((1, x.shape[1]), x.dtype)
    row = lax.broadcasted_iota(jnp.int32, (gh, 1), 0) % H
    zero = jnp.zeros((), x.dtype)
    xup = jnp.where(row == 0, zero, jnp.concatenate([zrow, x[:gh - 1]], 0))
    xdn = jnp.where(row == H - 1, zero, jnp.concatenate([x[1:], zrow], 0))
    kt = NPI * Cin
    ot = WT * Cout
    sums, sqs = [], []
    for t in range(W // WT):
        si = max(t - 1, 0) * WT * Cin
        wv = 0 if t == 0 else 1
        so = t * WT * Cout
        a = (jnp.dot(xup[:, si:si + kt], w_ref[wv, 0],
                     preferred_element_type=jnp.float32)
             + jnp.dot(x[:, si:si + kt], w_ref[wv, 1],
                       preferred_element_type=jnp.float32)
             + jnp.dot(xdn[:, si:si + kt], w_ref[wv, 2],
                       preferred_element_type=jnp.float32))
        y = a + b3_ref[:, so:so + ot]
        y1_ref[0, :, so:so + ot] = y
        sums.append(jnp.sum(y, axis=0, keepdims=True))
        sqs.append(jnp.sum(y * y, axis=0, keepdims=True))
    st_ref[0] = jnp.concatenate(
        [jnp.concatenate(sums, axis=1), jnp.concatenate(sqs, axis=1)], axis=0)


def _p2_kernel(x_ref, y1_ref, w1_ref, b1_ref, sc_ref, sh_ref, o_ref,
               *, W, Cin, Cout):
    x = x_ref[0]
    kt = NPI * Cin
    ot = WT * Cout
    for t in range(W // WT):
        si = max(t - 1, 0) * WT * Cin
        wv = 0 if t == 0 else 1
        so = t * WT * Cout
        y2 = jnp.dot(x[:, si:si + kt], w1_ref[wv],
                     preferred_element_type=jnp.float32)
        y1n = jnp.maximum(
            y1_ref[0, :, so:so + ot] * sc_ref[:, so:so + ot]
            + sh_ref[:, so:so + ot], 0.0)
        o_ref[0, :, so:so + ot] = jnp.maximum(
            y1n + y2 + b1_ref[:, so:so + ot], 0.0)


@jax.jit
def _forward(x_nchw, w3, b3, gamma, beta, w1, b1):
    N, Cin, H, W = x_nchw.shape
    Cout = w3.shape[-1]
    WCin, WCout = W * Cin, W * Cout
    P = N * H * W
    g = math.gcd(GIMG, N)
    ng = N // g
    gh = g * H

    x = jnp.transpose(x_nchw, (0, 2, 3, 1)).reshape(ng, gh, WCin)
    x = x.astype(jnp.bfloat16)
    w3f = w3.astype(jnp.float32)
    w1f = w1.astype(jnp.float32)
    w3b = jnp.stack([_band3_tile(w3f, 0), _band3_tile(w3f, WT)]
                    ).astype(jnp.bfloat16)
    w1b = jnp.stack([_band1_tile(w1f, 0), _band1_tile(w1f, WT)]
                    ).astype(jnp.bfloat16)
    b3t = _tile_lanes(b3, W).astype(jnp.float32)
    b1t = _tile_lanes(b1, W).astype(jnp.float32)

    cparams = pltpu.CompilerParams(
        dimension_semantics=("parallel",),
        vmem_limit_bytes=64 * 1024 * 1024,
    )

    kt = NPI * Cin
    ot = WT * Cout
    nt = W // WT

    flops1 = int(ng * nt * 3 * gh * kt * ot * 2 + N * 6 * H * WCout)
    bytes1 = int(2 * N * H * WCin + 2 * 2 * 3 * kt * ot
                 + 4 * (WCout + N * H * WCout + ng * 2 * WCout))
    y1, stats = pl.pallas_call(
        partial(_p1_kernel, H=H, W=W, Cin=Cin, Cout=Cout),
        grid=(ng,),
        in_specs=[
            pl.BlockSpec((1, gh, WCin), lambda n: (n, 0, 0)),
            pl.BlockSpec((2, 3, kt, ot), lambda n: (0, 0, 0, 0)),
            pl.BlockSpec((1, WCout), lambda n: (0, 0)),
        ],
        out_specs=(
            pl.BlockSpec((1, gh, WCout), lambda n: (n, 0, 0)),
            pl.BlockSpec((1, 2, WCout), lambda n: (n, 0, 0)),
        ),
        out_shape=(
            jax.ShapeDtypeStruct((ng, gh, WCout), jnp.float32),
            jax.ShapeDtypeStruct((ng, 2, WCout), jnp.float32),
        ),
        compiler_params=cparams,
        cost_estimate=pl.CostEstimate(flops=flops1, transcendentals=0,
                                      bytes_accessed=bytes1),
    )(x, w3b, b3t)

    s = stats.sum(axis=0).reshape(2, W, Cout).sum(axis=1)
    mean = s[0] / P
    var = s[1] / P - mean * mean
    scale = gamma.reshape(Cout) * lax.rsqrt(var + EPS)
    shift = beta.reshape(Cout) - mean * scale
    sc = _tile_lanes(scale, W).astype(jnp.float32)
    sh = _tile_lanes(shift, W).astype(jnp.float32)

    flops2 = int(ng * nt * gh * kt * ot * 2 + N * 6 * H * WCout)
    bytes2 = int(2 * N * H * WCin + 2 * 2 * kt * ot
                 + 4 * (N * H * WCout + 3 * WCout + N * H * WCout))
    out = pl.pallas_call(
        partial(_p2_kernel, W=W, Cin=Cin, Cout=Cout),
        grid=(ng,),
        in_specs=[
            pl.BlockSpec((1, gh, WCin), lambda n: (n, 0, 0)),
            pl.BlockSpec((1, gh, WCout), lambda n: (n, 0, 0)),
            pl.BlockSpec((2, kt, ot), lambda n: (0, 0, 0)),
            pl.BlockSpec((1, WCout), lambda n: (0, 0)),
            pl.BlockSpec((1, WCout), lambda n: (0, 0)),
            pl.BlockSpec((1, WCout), lambda n: (0, 0)),
        ],
        out_specs=pl.BlockSpec((1, gh, WCout), lambda n: (n, 0, 0)),
        out_shape=jax.ShapeDtypeStruct((ng, gh, WCout), jnp.float32),
        compiler_params=cparams,
        cost_estimate=pl.CostEstimate(flops=flops2, transcendentals=0,
                                      bytes_accessed=bytes2),
    )(x, y1, w1b, b1t, sc, sh)

    out = out.reshape(N, H, W, Cout)
    return jnp.transpose(out, (0, 3, 1, 2))


def kernel(x_nchw, w3, b3, gamma, beta, w1, b1):
    return _forward(x_nchw, w3, b3, gamma, beta, w1, b1)
